# baseline (speedup 1.0000x reference)
import numpy as np
import jax
import jax.numpy as jnp
from jax import lax
from jax.experimental import pallas as pl
from jax.experimental.pallas import tpu as pltpu
from jax.experimental.pallas import tpu_sc as plsc

DIM = 256
HEADS = 8
LEVELS = 4
POINTS = 4
DH = DIM // HEADS
N = 4
QN = 5440
HLP = HEADS * LEVELS * POINTS
R = N * QN * HEADS
UNITS = N * QN
NW = 32
UPW = QN // NW
RN = QN * HEADS

QBLK = 544
NQB = QN // QBLK

_SHAPES = np.array([[64, 64], [32, 32], [16, 16], [8, 8]], dtype=np.int32)
_LIDX = np.array([0, 4096, 5120, 5376], dtype=np.int32)

_lane = np.arange(HLP)
_l_of_lane = (_lane % 16) // 4
_h_of_lane = _lane // 16
_W_LANE_F = _SHAPES[_l_of_lane, 1].astype(np.float32).reshape(1, HLP)
_H_LANE_F = _SHAPES[_l_of_lane, 0].astype(np.float32).reshape(1, HLP)
_W_LANE_I = _SHAPES[_l_of_lane, 1].reshape(1, HLP)
_H_LANE_I = _SHAPES[_l_of_lane, 0].reshape(1, HLP)
_S_LANE_I = _LIDX[_l_of_lane].reshape(1, HLP)
_H_HEAD_I = _h_of_lane.astype(np.int32).reshape(1, HLP)
_SELXY = np.zeros((2, 2 * LEVELS, HLP), np.float32)
_SELXY[0, 2 * _l_of_lane, _lane] = 1.0
_SELXY[1, 2 * _l_of_lane + 1, _lane] = 1.0
_LANE_F = np.concatenate([_W_LANE_F, _H_LANE_F], axis=0)
_LANE_I = np.concatenate([_W_LANE_I, _H_LANE_I, _S_LANE_I, _H_HEAD_I], axis=0)

_PERM = np.arange(DIM).reshape(HEADS, 16, 2)
_PERM = np.concatenate([_PERM[:, :, 0], _PERM[:, :, 1]], axis=1).reshape(DIM)

_HP = jax.lax.Precision.HIGHEST


def _dot(a, b):
    return jnp.dot(a, b, preferred_element_type=jnp.float32, precision=_HP)


def _prep_body(nconst, q_ref, v_ref, p8_ref, wvt_ref, bv_ref, wsxt_ref, bsx_ref,
               wsyt_ref, bsy_ref, wat_ref, ba_ref, selxy_ref, lanef_ref,
               lanei_ref, vp_ref, idx_ref, wt_ref):
    n = nconst
    del n
    qb = q_ref[...]
    vb = v_ref[...]
    p8 = p8_ref[...]

    vp = _dot(vb, wvt_ref[...]) + bv_ref[0]
    for h in range(HEADS):
        vp_ref[h] = vp[:, h * DH:(h + 1) * DH]

    attn = _dot(qb, wat_ref[...]) + ba_ref[0]
    offx = _dot(qb, wsxt_ref[...]) + bsx_ref[0]
    offy = _dot(qb, wsyt_ref[...]) + bsy_ref[0]

    px = _dot(p8, selxy_ref[0])
    py = _dot(p8, selxy_ref[1])

    wf = lanef_ref[0:1, :]
    hf = lanef_ref[1:2, :]
    x = (px + offx / wf) * wf - 0.5
    y = (py + offy / hf) * hf - 0.5
    x0f = jnp.floor(x)
    y0f = jnp.floor(y)
    lx = x - x0f
    ly = y - y0f
    x0 = x0f.astype(jnp.int32)
    y0 = y0f.astype(jnp.int32)

    wi = lanei_ref[0:1, :]
    hi = lanei_ref[1:2, :]
    si = lanei_ref[2:3, :]
    hh = lanei_ref[3:4, :]

    zero = jnp.zeros_like(lx)
    xb = jnp.clip(x0, 0, wi - 2)
    yb = jnp.clip(y0, 0, hi - 2)
    wx0c = (1.0 - lx) * ((x0 >= 0) & (x0 <= wi - 1)).astype(jnp.float32)
    wx1c = lx * ((x0 >= -1) & (x0 <= wi - 2)).astype(jnp.float32)
    wy0c = (1.0 - ly) * ((y0 >= 0) & (y0 <= hi - 1)).astype(jnp.float32)
    wy1c = ly * ((y0 >= -1) & (y0 <= hi - 2)).astype(jnp.float32)
    wsx0 = jnp.where(x0 == xb, wx0c, zero) + jnp.where(x0 == xb - 1, wx1c, zero)
    wsx1 = jnp.where(x0 == xb + 1, wx0c, zero) + jnp.where(x0 == xb, wx1c, zero)
    wsy0 = jnp.where(y0 == yb, wy0c, zero) + jnp.where(y0 == yb - 1, wy1c, zero)
    wsy1 = jnp.where(y0 == yb + 1, wy0c, zero) + jnp.where(y0 == yb, wy1c, zero)

    idx_ref[...] = hh * QN + (si + yb * wi + xb)
    wt_ref[:, 0, :] = wsy0 * wsx0 * attn
    wt_ref[:, 1, :] = wsy0 * wsx1 * attn
    wt_ref[:, 2, :] = wsy1 * wsx0 * attn
    wt_ref[:, 3, :] = wsy1 * wsx1 * attn


def _prep(n, q, v, p8, wvt, bv, wsxt, bsx, wsyt, bsy, wat, ba, selxy, lanef, lanei):
    import functools
    full = lambda s: pl.BlockSpec(s, lambda i: (0,) * len(s))
    blk = lambda s: pl.BlockSpec(s, lambda i: (i,) + (0,) * (len(s) - 1))
    vblk = pl.BlockSpec((HEADS, QBLK, DH), lambda i: (0, i, 0))
    return pl.pallas_call(
        functools.partial(_prep_body, n),
        grid=(NQB,),
        in_specs=[
            blk((QBLK, DIM)),
            blk((QBLK, DIM)),
            blk((QBLK, 2 * LEVELS)),
            full((DIM, DIM)),
            full((1, DIM)),
            full((DIM, HLP)),
            full((1, HLP)),
            full((DIM, HLP)),
            full((1, HLP)),
            full((DIM, HLP)),
            full((1, HLP)),
            full((2, 2 * LEVELS, HLP)),
            full((2, HLP)),
            full((4, HLP)),
        ],
        out_specs=[
            vblk,
            blk((QBLK, HLP)),
            blk((QBLK, 4, HLP)),
        ],
        out_shape=[
            jax.ShapeDtypeStruct((HEADS, QN, DH), jnp.float32),
            jax.ShapeDtypeStruct((QN, HLP), jnp.int32),
            jax.ShapeDtypeStruct((QN, 4, HLP), jnp.float32),
        ],
    )(q, v, p8, wvt, bv, wsxt, bsx, wsyt, bsy, wat, ba, selxy, lanef, lanei)


def _t4_body(vp_ref, t4_ref):
    vpn = vp_ref[0]
    parts = []
    for l in range(LEVELS):
        h = int(_SHAPES[l, 0])
        w = int(_SHAPES[l, 1])
        s = int(_LIDX[l])
        hw = h * w
        a = vpn[s:s + hw]
        b = jnp.concatenate([vpn[s + 1:s + hw], vpn[s:s + 1]], 0)
        c = jnp.concatenate([vpn[s + w:s + hw], vpn[s:s + w]], 0)
        d = jnp.concatenate([vpn[s + w + 1:s + hw], vpn[s:s + w + 1]], 0)
        parts.append(jnp.concatenate([a, b, c, d], 1))
    patch = jnp.concatenate(parts, 0)
    def rne16(f):
        b = jax.lax.bitcast_convert_type(f, jnp.int32)
        r = b + 0x7FFF + jax.lax.shift_right_logical(b, 16) % 2
        return jax.lax.shift_right_logical(r, 16)
    lo = rne16(patch[:, 0:64])
    hi = rne16(patch[:, 64:128])
    t4_ref[...] = lo | jax.lax.shift_left(hi, 16)


def _t4(vp):
    return pl.pallas_call(
        _t4_body,
        grid=(HEADS,),
        in_specs=[pl.BlockSpec((1, QN, DH), lambda h: (h, 0, 0))],
        out_specs=pl.BlockSpec((QN, 2 * DH), lambda h: (h, 0)),
        out_shape=jax.ShapeDtypeStruct((HEADS * QN, 2 * DH), jnp.int32),
    )(vp)


def _sc_body(tbl0, ixh0, wth0, tbl1, ixh1, wth1, out0, out1,
             i0, i1, w0, w1, g0, g1, o0, o1,
             sg0, sg1, si0, si1, so0, so1):
    cid = lax.axis_index("c")
    sid = lax.axis_index("s")
    base = (sid * 2 + cid) * UPW
    bufs = ((i0, w0, g0, o0, sg0, si0, so0),
            (i1, w1, g1, o1, sg1, si1, so1))
    _run_phase(tbl0, ixh0, wth0, out0, base, bufs)
    _run_phase(tbl1, ixh1, wth1, out1, base, bufs)


def _run_phase(table_hbm, idx_hbm, wt_hbm, out_hbm, base, bufs):

    def fire_meta(t, b):
        iv, wv, _, _, _, si, _ = bufs[b]
        unit = base + t
        pltpu.async_copy(idx_hbm.at[unit], iv, si)
        pltpu.async_copy(wt_hbm.at[unit], wv, si)

    def wait_meta(b):
        iv, wv, _, _, _, si, _ = bufs[b]
        pltpu.make_async_copy(idx_hbm.at[0], iv, si).wait()
        pltpu.make_async_copy(wt_hbm.at[0], wv, si).wait()

    def fire_gather(b):
        iv, _, gv, _, sg, _, _ = bufs[b]
        pltpu.async_copy(table_hbm.at[iv], gv, sg)

    def wait_gather(b):
        iv, _, gv, _, sg, _, _ = bufs[b]
        pltpu.make_async_copy(table_hbm.at[iv], gv, sg).wait()

    def wait_out(b):
        _, _, _, ov, _, _, so = bufs[b]
        pltpu.make_async_copy(ov, out_hbm.at[pl.ds(0, HEADS)], so).wait()

    def compute(t, b):
        _, wv, gv, ov, _, _, so = bufs[b]
        mask = jnp.full((16,), -65536, jnp.int32)

        def expand(r):
            lo = plsc.bitcast(jax.lax.shift_left(r, 16), jnp.float32)
            hi = plsc.bitcast(r & mask, jnp.float32)
            return lo, hi

        for h in range(HEADS):
            def body(j, accs, h=h):
                a0, a1 = accs
                i = h * 16 + j
                iv16 = jnp.full((16,), i, jnp.int32)
                w00 = plsc.load_gather(wv, [jnp.full((16,), 0, jnp.int32), iv16])
                w01 = plsc.load_gather(wv, [jnp.full((16,), 1, jnp.int32), iv16])
                w10 = plsc.load_gather(wv, [jnp.full((16,), 2, jnp.int32), iv16])
                w11 = plsc.load_gather(wv, [jnp.full((16,), 3, jnp.int32), iv16])
                a0l, c0l = expand(gv[i, pl.ds(0, 16)])
                a1l, c1l = expand(gv[i, pl.ds(16, 16)])
                b0l, d0l = expand(gv[i, pl.ds(32, 16)])
                b1l, d1l = expand(gv[i, pl.ds(48, 16)])
                a0 = a0 + w00 * a0l + w10 * c0l + w01 * b0l + w11 * d0l
                a1 = a1 + w00 * a1l + w10 * c1l + w01 * b1l + w11 * d1l
                return (a0, a1)
            z = jnp.zeros((16,), jnp.float32)
            a0, a1 = lax.fori_loop(0, 16, body, (z, z))
            ov[h, pl.ds(0, 16)] = a0
            ov[h, pl.ds(16, 16)] = a1
        pltpu.async_copy(ov, out_hbm.at[pl.ds((base + t) * HEADS, HEADS)], so)

    fire_meta(0, 0)
    fire_meta(1, 1)
    wait_meta(0)
    fire_gather(0)

    @pl.loop(0, UPW // 2)
    def _pair(tt):
        for b in range(2):
            t = tt * 2 + b
            wait_gather(b)

            @pl.when(t + 1 < UPW)
            def _():
                wait_meta(1 - b)
                fire_gather(1 - b)

            @pl.when(t >= 2)
            def _():
                wait_out(b)
            compute(t, b)

            @pl.when(t + 2 < UPW)
            def _():
                fire_meta(t + 2, b)

    wait_out(0)
    wait_out(1)


def _sc_gather(table0, idx0, wt0, table1, idx1, wt1):
    kfn = pl.kernel(
        _sc_body,
        out_type=[jax.ShapeDtypeStruct((RN, DH), jnp.float32),
                  jax.ShapeDtypeStruct((RN, DH), jnp.float32)],
        mesh=plsc.VectorSubcoreMesh(core_axis_name="c", subcore_axis_name="s"),
        scratch_types=[
            pltpu.VMEM((HLP,), jnp.int32),
            pltpu.VMEM((HLP,), jnp.int32),
            pltpu.VMEM((4, HLP), jnp.float32),
            pltpu.VMEM((4, HLP), jnp.float32),
            pltpu.VMEM((HLP, 2 * DH), jnp.int32),
            pltpu.VMEM((HLP, 2 * DH), jnp.int32),
            pltpu.VMEM((HEADS, DH), jnp.float32),
            pltpu.VMEM((HEADS, DH), jnp.float32),
            pltpu.SemaphoreType.DMA,
            pltpu.SemaphoreType.DMA,
            pltpu.SemaphoreType.DMA,
            pltpu.SemaphoreType.DMA,
            pltpu.SemaphoreType.DMA,
            pltpu.SemaphoreType.DMA,
        ],
        compiler_params=pltpu.CompilerParams(
            needs_layout_passes=False, use_tc_tiling_on_sc=False),
    )
    return kfn(table0, idx0, wt0, table1, idx1, wt1)


def _proj_body(acc_ref, wot_ref, bo_ref, out_ref):
    out_ref[...] = _dot(acc_ref[...], wot_ref[...]) + bo_ref[0]


def _proj(acc, wot, bo):
    full = lambda s: pl.BlockSpec(s, lambda i: (0,) * len(s))
    blk = lambda s: pl.BlockSpec(s, lambda i: (i, 0))
    return pl.pallas_call(
        _proj_body,
        grid=(NQB,),
        in_specs=[blk((QBLK, DIM)), full((DIM, DIM)), full((1, DIM))],
        out_specs=blk((QBLK, DIM)),
        out_shape=jax.ShapeDtypeStruct((QN, DIM), jnp.float32),
    )(acc, wot, bo)


def kernel(q, p, v, shapes, level_index, Wv, bv, Ws, bs, Wa, ba, Wo, bo):
    p8 = p.reshape(N, QN, 2 * LEVELS)
    wvt = Wv.T
    wsxt, bsx = Ws[0::2].T, bs[0::2].reshape(1, HLP)
    wsyt, bsy = Ws[1::2].T, bs[1::2].reshape(1, HLP)
    wat, ba2 = Wa.T, ba.reshape(1, HLP)
    bv2 = bv.reshape(1, DIM)
    bo2 = bo.reshape(1, DIM)
    selxy = jnp.asarray(_SELXY)
    lanef = jnp.asarray(_LANE_F)
    lanei = jnp.asarray(_LANE_I)
    wot_perm = Wo.T
    outs = []
    for pair in range(N // 2):
        sets = []
        for n in (2 * pair, 2 * pair + 1):
            vp, idx, wt = _prep(n, q[n], v[n], p8[n], wvt, bv2, wsxt, bsx,
                                wsyt, bsy, wat, ba2, selxy, lanef, lanei)
            sets += [_t4(vp), idx, wt]
        acc0, acc1 = _sc_gather(*sets)
        outs.append(_proj(acc0.reshape(QN, DIM), wot_perm, bo2))
        outs.append(_proj(acc1.reshape(QN, DIM), wot_perm, bo2))
    return jnp.stack(outs)


# speedup vs baseline: 1.0239x; 1.0239x over previous
import numpy as np
import jax
import jax.numpy as jnp
from jax import lax
from jax.experimental import pallas as pl
from jax.experimental.pallas import tpu as pltpu
from jax.experimental.pallas import tpu_sc as plsc

DIM = 256
HEADS = 8
LEVELS = 4
POINTS = 4
DH = DIM // HEADS
N = 4
QN = 5440
HLP = HEADS * LEVELS * POINTS
R = N * QN * HEADS
UNITS = N * QN
NW = 32
UPW = QN // NW
RN = QN * HEADS

QBLK = 1088
NQB = QN // QBLK

_SHAPES = np.array([[64, 64], [32, 32], [16, 16], [8, 8]], dtype=np.int32)
_LIDX = np.array([0, 4096, 5120, 5376], dtype=np.int32)

_lane = np.arange(HLP)
_l_of_lane = (_lane % 16) // 4
_h_of_lane = _lane // 16
_W_LANE_F = _SHAPES[_l_of_lane, 1].astype(np.float32).reshape(1, HLP)
_H_LANE_F = _SHAPES[_l_of_lane, 0].astype(np.float32).reshape(1, HLP)
_W_LANE_I = _SHAPES[_l_of_lane, 1].reshape(1, HLP)
_H_LANE_I = _SHAPES[_l_of_lane, 0].reshape(1, HLP)
_S_LANE_I = _LIDX[_l_of_lane].reshape(1, HLP)
_H_HEAD_I = _h_of_lane.astype(np.int32).reshape(1, HLP)
_SELXY = np.zeros((2, 2 * LEVELS, HLP), np.float32)
_SELXY[0, 2 * _l_of_lane, _lane] = 1.0
_SELXY[1, 2 * _l_of_lane + 1, _lane] = 1.0
_LANE_F = np.concatenate([_W_LANE_F, _H_LANE_F], axis=0)
_LANE_I = np.concatenate([_W_LANE_I, _H_LANE_I, _S_LANE_I, _H_HEAD_I], axis=0)

_PERM = np.arange(DIM).reshape(HEADS, 16, 2)
_PERM = np.concatenate([_PERM[:, :, 0], _PERM[:, :, 1]], axis=1).reshape(DIM)

_HP = jax.lax.Precision.HIGHEST


def _dot(a, b):
    return jnp.dot(a, b, preferred_element_type=jnp.float32, precision=_HP)


def _prep_body(nconst, q_ref, v_ref, p8_ref, wvt_ref, bv_ref, wsxt_ref, bsx_ref,
               wsyt_ref, bsy_ref, wat_ref, ba_ref, selxy_ref, lanef_ref,
               lanei_ref, vp_ref, idx_ref, wt_ref):
    n = nconst
    del n
    qb = q_ref[...]
    vb = v_ref[...]
    p8 = p8_ref[...]

    vp = _dot(vb, wvt_ref[...]) + bv_ref[0]
    for h in range(HEADS):
        vp_ref[h] = vp[:, h * DH:(h + 1) * DH]

    attn = _dot(qb, wat_ref[...]) + ba_ref[0]
    offx = _dot(qb, wsxt_ref[...]) + bsx_ref[0]
    offy = _dot(qb, wsyt_ref[...]) + bsy_ref[0]

    px = _dot(p8, selxy_ref[0])
    py = _dot(p8, selxy_ref[1])

    wf = lanef_ref[0:1, :]
    hf = lanef_ref[1:2, :]
    x = (px + offx / wf) * wf - 0.5
    y = (py + offy / hf) * hf - 0.5
    x0f = jnp.floor(x)
    y0f = jnp.floor(y)
    lx = x - x0f
    ly = y - y0f
    x0 = x0f.astype(jnp.int32)
    y0 = y0f.astype(jnp.int32)

    wi = lanei_ref[0:1, :]
    hi = lanei_ref[1:2, :]
    si = lanei_ref[2:3, :]
    hh = lanei_ref[3:4, :]

    zero = jnp.zeros_like(lx)
    xb = jnp.clip(x0, 0, wi - 2)
    yb = jnp.clip(y0, 0, hi - 2)
    wx0c = (1.0 - lx) * ((x0 >= 0) & (x0 <= wi - 1)).astype(jnp.float32)
    wx1c = lx * ((x0 >= -1) & (x0 <= wi - 2)).astype(jnp.float32)
    wy0c = (1.0 - ly) * ((y0 >= 0) & (y0 <= hi - 1)).astype(jnp.float32)
    wy1c = ly * ((y0 >= -1) & (y0 <= hi - 2)).astype(jnp.float32)
    wsx0 = jnp.where(x0 == xb, wx0c, zero) + jnp.where(x0 == xb - 1, wx1c, zero)
    wsx1 = jnp.where(x0 == xb + 1, wx0c, zero) + jnp.where(x0 == xb, wx1c, zero)
    wsy0 = jnp.where(y0 == yb, wy0c, zero) + jnp.where(y0 == yb - 1, wy1c, zero)
    wsy1 = jnp.where(y0 == yb + 1, wy0c, zero) + jnp.where(y0 == yb, wy1c, zero)

    idx_ref[...] = hh * QN + (si + yb * wi + xb)
    wt_ref[:, 0, :] = wsy0 * wsx0 * attn
    wt_ref[:, 1, :] = wsy0 * wsx1 * attn
    wt_ref[:, 2, :] = wsy1 * wsx0 * attn
    wt_ref[:, 3, :] = wsy1 * wsx1 * attn


def _prep(n, q, v, p8, wvt, bv, wsxt, bsx, wsyt, bsy, wat, ba, selxy, lanef, lanei):
    import functools
    full = lambda s: pl.BlockSpec(s, lambda i: (0,) * len(s))
    blk = lambda s: pl.BlockSpec(s, lambda i: (i,) + (0,) * (len(s) - 1))
    vblk = pl.BlockSpec((HEADS, QBLK, DH), lambda i: (0, i, 0))
    return pl.pallas_call(
        functools.partial(_prep_body, n),
        grid=(NQB,),
        in_specs=[
            blk((QBLK, DIM)),
            blk((QBLK, DIM)),
            blk((QBLK, 2 * LEVELS)),
            full((DIM, DIM)),
            full((1, DIM)),
            full((DIM, HLP)),
            full((1, HLP)),
            full((DIM, HLP)),
            full((1, HLP)),
            full((DIM, HLP)),
            full((1, HLP)),
            full((2, 2 * LEVELS, HLP)),
            full((2, HLP)),
            full((4, HLP)),
        ],
        out_specs=[
            vblk,
            blk((QBLK, HLP)),
            blk((QBLK, 4, HLP)),
        ],
        out_shape=[
            jax.ShapeDtypeStruct((HEADS, QN, DH), jnp.float32),
            jax.ShapeDtypeStruct((QN, HLP), jnp.int32),
            jax.ShapeDtypeStruct((QN, 4, HLP), jnp.float32),
        ],
    )(q, v, p8, wvt, bv, wsxt, bsx, wsyt, bsy, wat, ba, selxy, lanef, lanei)


def _t4_body(vp_ref, t4_ref):
    vpn = vp_ref[0]
    parts = []
    for l in range(LEVELS):
        h = int(_SHAPES[l, 0])
        w = int(_SHAPES[l, 1])
        s = int(_LIDX[l])
        hw = h * w
        a = vpn[s:s + hw]
        b = jnp.concatenate([vpn[s + 1:s + hw], vpn[s:s + 1]], 0)
        c = jnp.concatenate([vpn[s + w:s + hw], vpn[s:s + w]], 0)
        d = jnp.concatenate([vpn[s + w + 1:s + hw], vpn[s:s + w + 1]], 0)
        parts.append(jnp.concatenate([a, b, c, d], 1))
    patch = jnp.concatenate(parts, 0)
    def rne16(f):
        b = jax.lax.bitcast_convert_type(f, jnp.int32)
        r = b + 0x7FFF + jax.lax.shift_right_logical(b, 16) % 2
        return jax.lax.shift_right_logical(r, 16)
    lo = rne16(patch[:, 0:64])
    hi = rne16(patch[:, 64:128])
    t4_ref[...] = lo | jax.lax.shift_left(hi, 16)


def _t4(vp):
    return pl.pallas_call(
        _t4_body,
        grid=(HEADS,),
        in_specs=[pl.BlockSpec((1, QN, DH), lambda h: (h, 0, 0))],
        out_specs=pl.BlockSpec((QN, 2 * DH), lambda h: (h, 0)),
        out_shape=jax.ShapeDtypeStruct((HEADS * QN, 2 * DH), jnp.int32),
    )(vp)


def _sc_body(table_hbm, idx_hbm, wt_hbm, out_hbm,
             i0, i1, w0, w1, g0, g1, o0, o1,
             sg0, sg1, si0, si1, so0, so1):
    cid = lax.axis_index("c")
    sid = lax.axis_index("s")
    base = (sid * 2 + cid) * UPW
    bufs = ((i0, w0, g0, o0, sg0, si0, so0),
            (i1, w1, g1, o1, sg1, si1, so1))

    def fire_meta(t, b):
        iv, wv, _, _, _, si, _ = bufs[b]
        unit = base + t
        pltpu.async_copy(idx_hbm.at[unit], iv, si)
        pltpu.async_copy(wt_hbm.at[unit], wv, si)

    def wait_meta(b):
        iv, wv, _, _, _, si, _ = bufs[b]
        pltpu.make_async_copy(idx_hbm.at[0], iv, si).wait()
        pltpu.make_async_copy(wt_hbm.at[0], wv, si).wait()

    def fire_gather(b):
        iv, _, gv, _, sg, _, _ = bufs[b]
        pltpu.async_copy(table_hbm.at[iv], gv, sg)

    def wait_gather(b):
        iv, _, gv, _, sg, _, _ = bufs[b]
        pltpu.make_async_copy(table_hbm.at[iv], gv, sg).wait()

    def wait_out(b):
        _, _, _, ov, _, _, so = bufs[b]
        pltpu.make_async_copy(ov, out_hbm.at[pl.ds(0, HEADS)], so).wait()

    def compute(t, b):
        _, wv, gv, ov, _, _, so = bufs[b]
        mask = jnp.full((16,), -65536, jnp.int32)

        def expand(r):
            lo = plsc.bitcast(jax.lax.shift_left(r, 16), jnp.float32)
            hi = plsc.bitcast(r & mask, jnp.float32)
            return lo, hi

        for h in range(HEADS):
            def body(j, accs, h=h):
                a0, a1 = accs
                i = h * 16 + j
                iv16 = jnp.full((16,), i, jnp.int32)
                w00 = plsc.load_gather(wv, [jnp.full((16,), 0, jnp.int32), iv16])
                w01 = plsc.load_gather(wv, [jnp.full((16,), 1, jnp.int32), iv16])
                w10 = plsc.load_gather(wv, [jnp.full((16,), 2, jnp.int32), iv16])
                w11 = plsc.load_gather(wv, [jnp.full((16,), 3, jnp.int32), iv16])
                a0l, c0l = expand(gv[i, pl.ds(0, 16)])
                a1l, c1l = expand(gv[i, pl.ds(16, 16)])
                b0l, d0l = expand(gv[i, pl.ds(32, 16)])
                b1l, d1l = expand(gv[i, pl.ds(48, 16)])
                a0 = a0 + w00 * a0l + w10 * c0l + w01 * b0l + w11 * d0l
                a1 = a1 + w00 * a1l + w10 * c1l + w01 * b1l + w11 * d1l
                return (a0, a1)
            z = jnp.zeros((16,), jnp.float32)
            a0, a1 = lax.fori_loop(0, 16, body, (z, z))
            ov[h, pl.ds(0, 16)] = a0
            ov[h, pl.ds(16, 16)] = a1
        pltpu.async_copy(ov, out_hbm.at[pl.ds((base + t) * HEADS, HEADS)], so)

    fire_meta(0, 0)
    fire_meta(1, 1)
    wait_meta(0)
    fire_gather(0)

    @pl.loop(0, UPW // 2)
    def _pair(tt):
        for b in range(2):
            t = tt * 2 + b
            wait_gather(b)

            @pl.when(t + 1 < UPW)
            def _():
                wait_meta(1 - b)
                fire_gather(1 - b)

            @pl.when(t >= 2)
            def _():
                wait_out(b)
            compute(t, b)

            @pl.when(t + 2 < UPW)
            def _():
                fire_meta(t + 2, b)

    wait_out(0)
    wait_out(1)


def _sc_gather(table, idx, wt):
    kfn = pl.kernel(
        _sc_body,
        out_type=jax.ShapeDtypeStruct((RN, DH), jnp.float32),
        mesh=plsc.VectorSubcoreMesh(core_axis_name="c", subcore_axis_name="s"),
        scratch_types=[
            pltpu.VMEM((HLP,), jnp.int32),
            pltpu.VMEM((HLP,), jnp.int32),
            pltpu.VMEM((4, HLP), jnp.float32),
            pltpu.VMEM((4, HLP), jnp.float32),
            pltpu.VMEM((HLP, 2 * DH), jnp.int32),
            pltpu.VMEM((HLP, 2 * DH), jnp.int32),
            pltpu.VMEM((HEADS, DH), jnp.float32),
            pltpu.VMEM((HEADS, DH), jnp.float32),
            pltpu.SemaphoreType.DMA,
            pltpu.SemaphoreType.DMA,
            pltpu.SemaphoreType.DMA,
            pltpu.SemaphoreType.DMA,
            pltpu.SemaphoreType.DMA,
            pltpu.SemaphoreType.DMA,
        ],
        compiler_params=pltpu.CompilerParams(
            needs_layout_passes=False, use_tc_tiling_on_sc=False),
    )
    return kfn(table, idx, wt)


def _proj_body(acc_ref, wot_ref, bo_ref, out_ref):
    out_ref[...] = _dot(acc_ref[...], wot_ref[...]) + bo_ref[0]


def _proj(acc, wot, bo):
    full = lambda s: pl.BlockSpec(s, lambda i: (0,) * len(s))
    blk = lambda s: pl.BlockSpec(s, lambda i: (i, 0))
    return pl.pallas_call(
        _proj_body,
        grid=(NQB,),
        in_specs=[blk((QBLK, DIM)), full((DIM, DIM)), full((1, DIM))],
        out_specs=blk((QBLK, DIM)),
        out_shape=jax.ShapeDtypeStruct((QN, DIM), jnp.float32),
    )(acc, wot, bo)


def kernel(q, p, v, shapes, level_index, Wv, bv, Ws, bs, Wa, ba, Wo, bo):
    p8 = p.reshape(N, QN, 2 * LEVELS)
    wvt = Wv.T
    wsxt, bsx = Ws[0::2].T, bs[0::2].reshape(1, HLP)
    wsyt, bsy = Ws[1::2].T, bs[1::2].reshape(1, HLP)
    wat, ba2 = Wa.T, ba.reshape(1, HLP)
    bv2 = bv.reshape(1, DIM)
    bo2 = bo.reshape(1, DIM)
    selxy = jnp.asarray(_SELXY)
    lanef = jnp.asarray(_LANE_F)
    lanei = jnp.asarray(_LANE_I)
    wot_perm = Wo.T
    outs = []
    for n in range(N):
        vp, idx, wt = _prep(n, q[n], v[n], p8[n], wvt, bv2, wsxt, bsx,
                            wsyt, bsy, wat, ba2, selxy, lanef, lanei)
        table = _t4(vp)
        acc = _sc_gather(table, idx, wt)
        outs.append(_proj(acc.reshape(QN, DIM), wot_perm, bo2))
    return jnp.stack(outs)


# speedup vs baseline: 1.0278x; 1.0038x over previous
import numpy as np
import jax
import jax.numpy as jnp
from jax import lax
from jax.experimental import pallas as pl
from jax.experimental.pallas import tpu as pltpu
from jax.experimental.pallas import tpu_sc as plsc

DIM = 256
HEADS = 8
LEVELS = 4
POINTS = 4
DH = DIM // HEADS
N = 4
QN = 5440
HLP = HEADS * LEVELS * POINTS
R = N * QN * HEADS
UNITS = N * QN
NW = 32
UPW = QN // NW
RN = QN * HEADS

QBLK = 544
NQB = QN // QBLK

_SHAPES = np.array([[64, 64], [32, 32], [16, 16], [8, 8]], dtype=np.int32)
_LIDX = np.array([0, 4096, 5120, 5376], dtype=np.int32)

_lane = np.arange(HLP)
_l_of_lane = (_lane % 16) // 4
_h_of_lane = _lane // 16
_W_LANE_F = _SHAPES[_l_of_lane, 1].astype(np.float32).reshape(1, HLP)
_H_LANE_F = _SHAPES[_l_of_lane, 0].astype(np.float32).reshape(1, HLP)
_W_LANE_I = _SHAPES[_l_of_lane, 1].reshape(1, HLP)
_H_LANE_I = _SHAPES[_l_of_lane, 0].reshape(1, HLP)
_S_LANE_I = _LIDX[_l_of_lane].reshape(1, HLP)
_H_HEAD_I = _h_of_lane.astype(np.int32).reshape(1, HLP)
_SELXY = np.zeros((2, 2 * LEVELS, HLP), np.float32)
_SELXY[0, 2 * _l_of_lane, _lane] = 1.0
_SELXY[1, 2 * _l_of_lane + 1, _lane] = 1.0
_LANE_F = np.concatenate([_W_LANE_F, _H_LANE_F], axis=0)
_LANE_I = np.concatenate([_W_LANE_I, _H_LANE_I, _S_LANE_I, _H_HEAD_I], axis=0)

_PERM = np.arange(DIM).reshape(HEADS, 16, 2)
_PERM = np.concatenate([_PERM[:, :, 0], _PERM[:, :, 1]], axis=1).reshape(DIM)

_HP = jax.lax.Precision.HIGHEST


def _dot(a, b):
    return jnp.dot(a, b, preferred_element_type=jnp.float32, precision=_HP)


def _prep_body(nconst, q_ref, v_ref, p8_ref, wvt_ref, bv_ref, wsxt_ref, bsx_ref,
               wsyt_ref, bsy_ref, wat_ref, ba_ref, selxy_ref, lanef_ref,
               lanei_ref, vp_ref, idx_ref, wt_ref):
    n = nconst
    del n
    qb = q_ref[...]
    vb = v_ref[...]
    p8 = p8_ref[...]

    vp = _dot(vb, wvt_ref[...]) + bv_ref[0]
    for h in range(HEADS):
        vp_ref[h] = vp[:, h * DH:(h + 1) * DH]

    attn = _dot(qb, wat_ref[...]) + ba_ref[0]
    offx = _dot(qb, wsxt_ref[...]) + bsx_ref[0]
    offy = _dot(qb, wsyt_ref[...]) + bsy_ref[0]

    px = _dot(p8, selxy_ref[0])
    py = _dot(p8, selxy_ref[1])

    wf = lanef_ref[0:1, :]
    hf = lanef_ref[1:2, :]
    x = (px + offx / wf) * wf - 0.5
    y = (py + offy / hf) * hf - 0.5
    x0f = jnp.floor(x)
    y0f = jnp.floor(y)
    lx = x - x0f
    ly = y - y0f
    x0 = x0f.astype(jnp.int32)
    y0 = y0f.astype(jnp.int32)

    wi = lanei_ref[0:1, :]
    hi = lanei_ref[1:2, :]
    si = lanei_ref[2:3, :]
    hh = lanei_ref[3:4, :]

    zero = jnp.zeros_like(lx)
    xb = jnp.clip(x0, 0, wi - 2)
    yb = jnp.clip(y0, 0, hi - 2)
    wx0c = (1.0 - lx) * ((x0 >= 0) & (x0 <= wi - 1)).astype(jnp.float32)
    wx1c = lx * ((x0 >= -1) & (x0 <= wi - 2)).astype(jnp.float32)
    wy0c = (1.0 - ly) * ((y0 >= 0) & (y0 <= hi - 1)).astype(jnp.float32)
    wy1c = ly * ((y0 >= -1) & (y0 <= hi - 2)).astype(jnp.float32)
    wsx0 = jnp.where(x0 == xb, wx0c, zero) + jnp.where(x0 == xb - 1, wx1c, zero)
    wsx1 = jnp.where(x0 == xb + 1, wx0c, zero) + jnp.where(x0 == xb, wx1c, zero)
    wsy0 = jnp.where(y0 == yb, wy0c, zero) + jnp.where(y0 == yb - 1, wy1c, zero)
    wsy1 = jnp.where(y0 == yb + 1, wy0c, zero) + jnp.where(y0 == yb, wy1c, zero)

    idx_ref[...] = hh * QN + (si + yb * wi + xb)
    wt_ref[:, 0, :] = wsy0 * wsx0 * attn
    wt_ref[:, 1, :] = wsy0 * wsx1 * attn
    wt_ref[:, 2, :] = wsy1 * wsx0 * attn
    wt_ref[:, 3, :] = wsy1 * wsx1 * attn


def _prep(n, q, v, p8, wvt, bv, wsxt, bsx, wsyt, bsy, wat, ba, selxy, lanef, lanei):
    import functools
    full = lambda s: pl.BlockSpec(s, lambda i: (0,) * len(s))
    blk = lambda s: pl.BlockSpec(s, lambda i: (i,) + (0,) * (len(s) - 1))
    vblk = pl.BlockSpec((HEADS, QBLK, DH), lambda i: (0, i, 0))
    return pl.pallas_call(
        functools.partial(_prep_body, n),
        grid=(NQB,),
        in_specs=[
            blk((QBLK, DIM)),
            blk((QBLK, DIM)),
            blk((QBLK, 2 * LEVELS)),
            full((DIM, DIM)),
            full((1, DIM)),
            full((DIM, HLP)),
            full((1, HLP)),
            full((DIM, HLP)),
            full((1, HLP)),
            full((DIM, HLP)),
            full((1, HLP)),
            full((2, 2 * LEVELS, HLP)),
            full((2, HLP)),
            full((4, HLP)),
        ],
        out_specs=[
            vblk,
            blk((QBLK, HLP)),
            blk((QBLK, 4, HLP)),
        ],
        out_shape=[
            jax.ShapeDtypeStruct((HEADS, QN, DH), jnp.float32),
            jax.ShapeDtypeStruct((QN, HLP), jnp.int32),
            jax.ShapeDtypeStruct((QN, 4, HLP), jnp.float32),
        ],
    )(q, v, p8, wvt, bv, wsxt, bsx, wsyt, bsy, wat, ba, selxy, lanef, lanei)


def _t4_body(vp_ref, t4_ref):
    vpn = vp_ref[0]
    parts = []
    for l in range(LEVELS):
        h = int(_SHAPES[l, 0])
        w = int(_SHAPES[l, 1])
        s = int(_LIDX[l])
        hw = h * w
        a = vpn[s:s + hw]
        b = jnp.concatenate([vpn[s + 1:s + hw], vpn[s:s + 1]], 0)
        c = jnp.concatenate([vpn[s + w:s + hw], vpn[s:s + w]], 0)
        d = jnp.concatenate([vpn[s + w + 1:s + hw], vpn[s:s + w + 1]], 0)
        parts.append(jnp.concatenate([a, b, c, d], 1))
    patch = jnp.concatenate(parts, 0)
    def rne16(f):
        b = jax.lax.bitcast_convert_type(f, jnp.int32)
        r = b + 0x7FFF + jax.lax.shift_right_logical(b, 16) % 2
        return jax.lax.shift_right_logical(r, 16)
    lo = rne16(patch[:, 0:64])
    hi = rne16(patch[:, 64:128])
    t4_ref[...] = lo | jax.lax.shift_left(hi, 16)


def _t4(vp):
    return pl.pallas_call(
        _t4_body,
        grid=(HEADS,),
        in_specs=[pl.BlockSpec((1, QN, DH), lambda h: (h, 0, 0))],
        out_specs=pl.BlockSpec((QN, 2 * DH), lambda h: (h, 0)),
        out_shape=jax.ShapeDtypeStruct((HEADS * QN, 2 * DH), jnp.int32),
    )(vp)


def _sc_body(table_hbm, idx_hbm, wt_hbm, out_hbm,
             i0, i1, w0, w1, g0, g1, o0, o1,
             sg0, sg1, si0, si1, so0, so1):
    cid = lax.axis_index("c")
    sid = lax.axis_index("s")
    base = (sid * 2 + cid) * UPW
    bufs = ((i0, w0, g0, o0, sg0, si0, so0),
            (i1, w1, g1, o1, sg1, si1, so1))

    def fire_meta(t, b):
        iv, wv, _, _, _, si, _ = bufs[b]
        unit = base + t
        pltpu.async_copy(idx_hbm.at[unit], iv, si)
        pltpu.async_copy(wt_hbm.at[unit], wv, si)

    def wait_meta(b):
        iv, wv, _, _, _, si, _ = bufs[b]
        pltpu.make_async_copy(idx_hbm.at[0], iv, si).wait()
        pltpu.make_async_copy(wt_hbm.at[0], wv, si).wait()

    def fire_gather(b):
        iv, _, gv, _, sg, _, _ = bufs[b]
        pltpu.async_copy(table_hbm.at[iv], gv, sg)

    def wait_gather(b):
        iv, _, gv, _, sg, _, _ = bufs[b]
        pltpu.make_async_copy(table_hbm.at[iv], gv, sg).wait()

    def wait_out(b):
        _, _, _, ov, _, _, so = bufs[b]
        pltpu.make_async_copy(ov, out_hbm.at[pl.ds(0, HEADS)], so).wait()

    def compute(t, b):
        _, wv, gv, ov, _, _, so = bufs[b]
        mask = jnp.full((16,), -65536, jnp.int32)

        def expand(r):
            lo = plsc.bitcast(jax.lax.shift_left(r, 16), jnp.float32)
            hi = plsc.bitcast(r & mask, jnp.float32)
            return lo, hi

        for h in range(HEADS):
            def body(j, accs, h=h):
                a0, a1 = accs
                i = h * 16 + j
                iv16 = jnp.full((16,), i, jnp.int32)
                w00 = plsc.load_gather(wv, [jnp.full((16,), 0, jnp.int32), iv16])
                w01 = plsc.load_gather(wv, [jnp.full((16,), 1, jnp.int32), iv16])
                w10 = plsc.load_gather(wv, [jnp.full((16,), 2, jnp.int32), iv16])
                w11 = plsc.load_gather(wv, [jnp.full((16,), 3, jnp.int32), iv16])
                a0l, c0l = expand(gv[i, pl.ds(0, 16)])
                a1l, c1l = expand(gv[i, pl.ds(16, 16)])
                b0l, d0l = expand(gv[i, pl.ds(32, 16)])
                b1l, d1l = expand(gv[i, pl.ds(48, 16)])
                a0 = a0 + w00 * a0l + w10 * c0l + w01 * b0l + w11 * d0l
                a1 = a1 + w00 * a1l + w10 * c1l + w01 * b1l + w11 * d1l
                return (a0, a1)
            z = jnp.zeros((16,), jnp.float32)
            a0, a1 = lax.fori_loop(0, 16, body, (z, z))
            ov[h, pl.ds(0, 16)] = a0
            ov[h, pl.ds(16, 16)] = a1
        pltpu.async_copy(ov, out_hbm.at[pl.ds((base + t) * HEADS, HEADS)], so)

    fire_meta(0, 0)
    fire_meta(1, 1)
    wait_meta(0)
    fire_gather(0)

    @pl.loop(0, UPW // 2)
    def _pair(tt):
        for b in range(2):
            t = tt * 2 + b
            wait_gather(b)

            @pl.when(t + 1 < UPW)
            def _():
                wait_meta(1 - b)
                fire_gather(1 - b)

            @pl.when(t >= 2)
            def _():
                wait_out(b)
            compute(t, b)

            @pl.when(t + 2 < UPW)
            def _():
                fire_meta(t + 2, b)

    wait_out(0)
    wait_out(1)


def _sc_gather(table, idx, wt):
    kfn = pl.kernel(
        _sc_body,
        out_type=jax.ShapeDtypeStruct((RN, DH), jnp.float32),
        mesh=plsc.VectorSubcoreMesh(core_axis_name="c", subcore_axis_name="s"),
        scratch_types=[
            pltpu.VMEM((HLP,), jnp.int32),
            pltpu.VMEM((HLP,), jnp.int32),
            pltpu.VMEM((4, HLP), jnp.float32),
            pltpu.VMEM((4, HLP), jnp.float32),
            pltpu.VMEM((HLP, 2 * DH), jnp.int32),
            pltpu.VMEM((HLP, 2 * DH), jnp.int32),
            pltpu.VMEM((HEADS, DH), jnp.float32),
            pltpu.VMEM((HEADS, DH), jnp.float32),
            pltpu.SemaphoreType.DMA,
            pltpu.SemaphoreType.DMA,
            pltpu.SemaphoreType.DMA,
            pltpu.SemaphoreType.DMA,
            pltpu.SemaphoreType.DMA,
            pltpu.SemaphoreType.DMA,
        ],
        compiler_params=pltpu.CompilerParams(
            needs_layout_passes=False, use_tc_tiling_on_sc=False),
    )
    return kfn(table, idx, wt)


def _proj_body(acc_ref, wot_ref, bo_ref, out_ref):
    out_ref[...] = _dot(acc_ref[...], wot_ref[...]) + bo_ref[0]


def _proj(acc, wot, bo):
    full = lambda s: pl.BlockSpec(s, lambda i: (0,) * len(s))
    blk = lambda s: pl.BlockSpec(s, lambda i: (i, 0))
    return pl.pallas_call(
        _proj_body,
        grid=(NQB,),
        in_specs=[blk((QBLK, DIM)), full((DIM, DIM)), full((1, DIM))],
        out_specs=blk((QBLK, DIM)),
        out_shape=jax.ShapeDtypeStruct((QN, DIM), jnp.float32),
    )(acc, wot, bo)


def kernel(q, p, v, shapes, level_index, Wv, bv, Ws, bs, Wa, ba, Wo, bo):
    p8 = p.reshape(N, QN, 2 * LEVELS)
    wvt = Wv.T
    wsxt, bsx = Ws[0::2].T, bs[0::2].reshape(1, HLP)
    wsyt, bsy = Ws[1::2].T, bs[1::2].reshape(1, HLP)
    wat, ba2 = Wa.T, ba.reshape(1, HLP)
    bv2 = bv.reshape(1, DIM)
    bo2 = bo.reshape(1, DIM)
    selxy = jnp.asarray(_SELXY)
    lanef = jnp.asarray(_LANE_F)
    lanei = jnp.asarray(_LANE_I)
    wot_perm = Wo.T
    outs = []
    for n in range(N):
        vp, idx, wt = _prep(n, q[n], v[n], p8[n], wvt, bv2, wsxt, bsx,
                            wsyt, bsy, wat, ba2, selxy, lanef, lanei)
        table = _t4(vp)
        acc = _sc_gather(table, idx, wt)
        outs.append(_proj(acc.reshape(QN, DIM), wot_perm, bo2))
    return jnp.stack(outs)


# speedup vs baseline: 1.0286x; 1.0007x over previous
import numpy as np
import jax
import jax.numpy as jnp
from jax import lax
from jax.experimental import pallas as pl
from jax.experimental.pallas import tpu as pltpu
from jax.experimental.pallas import tpu_sc as plsc

DIM = 256
HEADS = 8
LEVELS = 4
POINTS = 4
DH = DIM // HEADS
N = 4
QN = 5440
HLP = HEADS * LEVELS * POINTS
NW = 32
UPW = QN // NW
RN = QN * HEADS

QBLK = 544
NQB = QN // QBLK

_SHAPES = np.array([[64, 64], [32, 32], [16, 16], [8, 8]], dtype=np.int32)
_LIDX = np.array([0, 4096, 5120, 5376], dtype=np.int32)

_lane = np.arange(HLP)
_l_of_lane = (_lane % 16) // 4
_h_of_lane = _lane // 16
_W_LANE_F = _SHAPES[_l_of_lane, 1].astype(np.float32).reshape(1, HLP)
_H_LANE_F = _SHAPES[_l_of_lane, 0].astype(np.float32).reshape(1, HLP)
_W_LANE_I = _SHAPES[_l_of_lane, 1].reshape(1, HLP)
_H_LANE_I = _SHAPES[_l_of_lane, 0].reshape(1, HLP)
_S_LANE_I = _LIDX[_l_of_lane].reshape(1, HLP)
_H_HEAD_I = _h_of_lane.astype(np.int32).reshape(1, HLP)
_SELXY = np.zeros((2, 2 * LEVELS, HLP), np.float32)
_SELXY[0, 2 * _l_of_lane, _lane] = 1.0
_SELXY[1, 2 * _l_of_lane + 1, _lane] = 1.0
_LANE_F = np.concatenate([_W_LANE_F, _H_LANE_F], axis=0)
_LANE_I = np.concatenate([_W_LANE_I, _H_LANE_I, _S_LANE_I, _H_HEAD_I], axis=0)

_HP = jax.lax.Precision.HIGHEST


def _dot(a, b):
    return jnp.dot(a, b, preferred_element_type=jnp.float32, precision=_HP)


def _prep_body(nconst, q_ref, v_ref, p8_ref, wvt_ref, bv_ref, wsxt_ref, bsx_ref,
               wsyt_ref, bsy_ref, wat_ref, ba_ref, selxy_ref, lanef_ref,
               lanei_ref, vp_ref, idx_ref, wt_ref):
    n = nconst
    del n
    qb = q_ref[...]
    vb = v_ref[...]
    p8 = p8_ref[...]

    vp = _dot(vb, wvt_ref[...]) + bv_ref[0]
    for h in range(HEADS):
        vp_ref[h] = vp[:, h * DH:(h + 1) * DH]

    attn = _dot(qb, wat_ref[...]) + ba_ref[0]
    offx = _dot(qb, wsxt_ref[...]) + bsx_ref[0]
    offy = _dot(qb, wsyt_ref[...]) + bsy_ref[0]

    px = _dot(p8, selxy_ref[0])
    py = _dot(p8, selxy_ref[1])

    wf = lanef_ref[0:1, :]
    hf = lanef_ref[1:2, :]
    x = (px + offx / wf) * wf - 0.5
    y = (py + offy / hf) * hf - 0.5
    x0f = jnp.floor(x)
    y0f = jnp.floor(y)
    lx = x - x0f
    ly = y - y0f
    x0 = x0f.astype(jnp.int32)
    y0 = y0f.astype(jnp.int32)

    wi = lanei_ref[0:1, :]
    hi = lanei_ref[1:2, :]
    si = lanei_ref[2:3, :]
    hh = lanei_ref[3:4, :]

    zero = jnp.zeros_like(lx)
    xb = jnp.clip(x0, 0, wi - 2)
    yb = jnp.clip(y0, 0, hi - 2)
    wx0c = (1.0 - lx) * ((x0 >= 0) & (x0 <= wi - 1)).astype(jnp.float32)
    wx1c = lx * ((x0 >= -1) & (x0 <= wi - 2)).astype(jnp.float32)
    wy0c = (1.0 - ly) * ((y0 >= 0) & (y0 <= hi - 1)).astype(jnp.float32)
    wy1c = ly * ((y0 >= -1) & (y0 <= hi - 2)).astype(jnp.float32)
    wsx0 = jnp.where(x0 == xb, wx0c, zero) + jnp.where(x0 == xb - 1, wx1c, zero)
    wsx1 = jnp.where(x0 == xb + 1, wx0c, zero) + jnp.where(x0 == xb, wx1c, zero)
    wsy0 = jnp.where(y0 == yb, wy0c, zero) + jnp.where(y0 == yb - 1, wy1c, zero)
    wsy1 = jnp.where(y0 == yb + 1, wy0c, zero) + jnp.where(y0 == yb, wy1c, zero)

    idx_ref[...] = hh * QN + (si + yb * wi + xb)
    wt_ref[:, 0, :] = wsy0 * wsx0 * attn
    wt_ref[:, 1, :] = wsy0 * wsx1 * attn
    wt_ref[:, 2, :] = wsy1 * wsx0 * attn
    wt_ref[:, 3, :] = wsy1 * wsx1 * attn


def _prep(n, q, v, p8, wvt, bv, wsxt, bsx, wsyt, bsy, wat, ba, selxy, lanef, lanei):
    import functools
    full = lambda s: pl.BlockSpec(s, lambda i: (0,) * len(s))
    blk = lambda s: pl.BlockSpec(s, lambda i: (i,) + (0,) * (len(s) - 1))
    vblk = pl.BlockSpec((HEADS, QBLK, DH), lambda i: (0, i, 0))
    return pl.pallas_call(
        functools.partial(_prep_body, n),
        grid=(NQB,),
        in_specs=[
            blk((QBLK, DIM)),
            blk((QBLK, DIM)),
            blk((QBLK, 2 * LEVELS)),
            full((DIM, DIM)),
            full((1, DIM)),
            full((DIM, HLP)),
            full((1, HLP)),
            full((DIM, HLP)),
            full((1, HLP)),
            full((DIM, HLP)),
            full((1, HLP)),
            full((2, 2 * LEVELS, HLP)),
            full((2, HLP)),
            full((4, HLP)),
        ],
        out_specs=[
            vblk,
            blk((QBLK, HLP)),
            blk((QBLK, 4, HLP)),
        ],
        out_shape=[
            jax.ShapeDtypeStruct((HEADS, QN, DH), jnp.float32),
            jax.ShapeDtypeStruct((QN, HLP), jnp.int32),
            jax.ShapeDtypeStruct((QN, 4, HLP), jnp.float32),
        ],
    )(q, v, p8, wvt, bv, wsxt, bsx, wsyt, bsy, wat, ba, selxy, lanef, lanei)


def _t4_body(vp_ref, t4_ref):
    vpn = vp_ref[0]
    parts = []
    for l in range(LEVELS):
        h = int(_SHAPES[l, 0])
        w = int(_SHAPES[l, 1])
        s = int(_LIDX[l])
        hw = h * w
        a = vpn[s:s + hw]
        b = jnp.concatenate([vpn[s + 1:s + hw], vpn[s:s + 1]], 0)
        c = jnp.concatenate([vpn[s + w:s + hw], vpn[s:s + w]], 0)
        d = jnp.concatenate([vpn[s + w + 1:s + hw], vpn[s:s + w + 1]], 0)
        parts.append(jnp.concatenate([a, b, c, d], 1))
    patch = jnp.concatenate(parts, 0)
    def rne16(f):
        b = jax.lax.bitcast_convert_type(f, jnp.int32)
        r = b + 0x7FFF + jax.lax.shift_right_logical(b, 16) % 2
        return jax.lax.shift_right_logical(r, 16)
    lo = rne16(patch[:, 0:64])
    hi = rne16(patch[:, 64:128])
    t4_ref[...] = lo | jax.lax.shift_left(hi, 16)


def _t4(vp):
    return pl.pallas_call(
        _t4_body,
        grid=(HEADS,),
        in_specs=[pl.BlockSpec((1, QN, DH), lambda h: (h, 0, 0))],
        out_specs=pl.BlockSpec((QN, 2 * DH), lambda h: (h, 0)),
        out_shape=jax.ShapeDtypeStruct((HEADS * QN, 2 * DH), jnp.int32),
    )(vp)


def _sc_body(table_hbm, idx_hbm, wt_hbm, out_hbm,
             i0, i1, w0, w1, g0, g1, o0, o1,
             sg0, sg1, si0, si1, so0, so1):
    cid = lax.axis_index("c")
    sid = lax.axis_index("s")
    base = (sid * 2 + cid) * UPW
    bufs = ((i0, w0, g0, o0, sg0, si0, so0),
            (i1, w1, g1, o1, sg1, si1, so1))

    def fire_meta(t, b):
        iv, wv, _, _, _, si, _ = bufs[b]
        unit = base + t
        pltpu.async_copy(idx_hbm.at[unit], iv, si)
        pltpu.async_copy(wt_hbm.at[unit], wv, si)

    def wait_meta(b):
        iv, wv, _, _, _, si, _ = bufs[b]
        pltpu.make_async_copy(idx_hbm.at[0], iv, si).wait()
        pltpu.make_async_copy(wt_hbm.at[0], wv, si).wait()

    def fire_gather(b):
        iv, _, gv, _, sg, _, _ = bufs[b]
        pltpu.async_copy(table_hbm.at[iv], gv, sg)

    def wait_gather(b):
        iv, _, gv, _, sg, _, _ = bufs[b]
        pltpu.make_async_copy(table_hbm.at[iv], gv, sg).wait()

    def wait_out(b):
        _, _, _, ov, _, _, so = bufs[b]
        pltpu.make_async_copy(ov, out_hbm.at[pl.ds(0, HEADS)], so).wait()

    def compute(t, b):
        _, wv, gv, ov, _, _, so = bufs[b]
        mask = jnp.full((16,), -65536, jnp.int32)

        def expand(r):
            lo = plsc.bitcast(jax.lax.shift_left(r, 16), jnp.float32)
            hi = plsc.bitcast(r & mask, jnp.float32)
            return lo, hi

        for h in range(HEADS):
            def body(j, accs, h=h):
                a0, a1 = accs
                i = h * 16 + j
                iv16 = jnp.full((16,), i, jnp.int32)
                w00 = plsc.load_gather(wv, [jnp.full((16,), 0, jnp.int32), iv16])
                w01 = plsc.load_gather(wv, [jnp.full((16,), 1, jnp.int32), iv16])
                w10 = plsc.load_gather(wv, [jnp.full((16,), 2, jnp.int32), iv16])
                w11 = plsc.load_gather(wv, [jnp.full((16,), 3, jnp.int32), iv16])
                a0l, c0l = expand(gv[i, pl.ds(0, 16)])
                a1l, c1l = expand(gv[i, pl.ds(16, 16)])
                b0l, d0l = expand(gv[i, pl.ds(32, 16)])
                b1l, d1l = expand(gv[i, pl.ds(48, 16)])
                a0 = a0 + w00 * a0l + w10 * c0l + w01 * b0l + w11 * d0l
                a1 = a1 + w00 * a1l + w10 * c1l + w01 * b1l + w11 * d1l
                return (a0, a1)
            z = jnp.zeros((16,), jnp.float32)
            a0, a1 = lax.fori_loop(0, 16, body, (z, z))
            ov[h, pl.ds(0, 16)] = a0
            ov[h, pl.ds(16, 16)] = a1
        pltpu.async_copy(ov, out_hbm.at[pl.ds((base + t) * HEADS, HEADS)], so)

    fire_meta(0, 0)
    fire_meta(1, 1)
    wait_meta(0)
    fire_gather(0)

    @pl.loop(0, UPW // 2)
    def _pair(tt):
        for b in range(2):
            t = tt * 2 + b
            wait_gather(b)

            @pl.when(t + 1 < UPW)
            def _():
                wait_meta(1 - b)
                fire_gather(1 - b)

            @pl.when(t >= 2)
            def _():
                wait_out(b)
            compute(t, b)

            @pl.when(t + 2 < UPW)
            def _():
                fire_meta(t + 2, b)

    wait_out(0)
    wait_out(1)


def _sc_gather(table, idx, wt):
    kfn = pl.kernel(
        _sc_body,
        out_type=jax.ShapeDtypeStruct((RN, DH), jnp.float32),
        mesh=plsc.VectorSubcoreMesh(core_axis_name="c", subcore_axis_name="s"),
        scratch_types=[
            pltpu.VMEM((HLP,), jnp.int32),
            pltpu.VMEM((HLP,), jnp.int32),
            pltpu.VMEM((4, HLP), jnp.float32),
            pltpu.VMEM((4, HLP), jnp.float32),
            pltpu.VMEM((HLP, 2 * DH), jnp.int32),
            pltpu.VMEM((HLP, 2 * DH), jnp.int32),
            pltpu.VMEM((HEADS, DH), jnp.float32),
            pltpu.VMEM((HEADS, DH), jnp.float32),
            pltpu.SemaphoreType.DMA,
            pltpu.SemaphoreType.DMA,
            pltpu.SemaphoreType.DMA,
            pltpu.SemaphoreType.DMA,
            pltpu.SemaphoreType.DMA,
            pltpu.SemaphoreType.DMA,
        ],
        compiler_params=pltpu.CompilerParams(
            needs_layout_passes=False, use_tc_tiling_on_sc=False),
    )
    return kfn(table, idx, wt)


def _proj_body(acc_ref, wot_ref, bo_ref, out_ref):
    out_ref[...] = _dot(acc_ref[...], wot_ref[...]) + bo_ref[0]


def _proj(acc, wot, bo):
    full = lambda s: pl.BlockSpec(s, lambda i: (0,) * len(s))
    blk = lambda s: pl.BlockSpec(s, lambda i: (i, 0))
    return pl.pallas_call(
        _proj_body,
        grid=(NQB,),
        in_specs=[blk((QBLK, DIM)), full((DIM, DIM)), full((1, DIM))],
        out_specs=blk((QBLK, DIM)),
        out_shape=jax.ShapeDtypeStruct((QN, DIM), jnp.float32),
    )(acc, wot, bo)


def kernel(q, p, v, shapes, level_index, Wv, bv, Ws, bs, Wa, ba, Wo, bo):
    p8 = p.reshape(N, QN, 2 * LEVELS)
    wvt = Wv.T
    wsxt, bsx = Ws[0::2].T, bs[0::2].reshape(1, HLP)
    wsyt, bsy = Ws[1::2].T, bs[1::2].reshape(1, HLP)
    wat, ba2 = Wa.T, ba.reshape(1, HLP)
    bv2 = bv.reshape(1, DIM)
    bo2 = bo.reshape(1, DIM)
    selxy = jnp.asarray(_SELXY)
    lanef = jnp.asarray(_LANE_F)
    lanei = jnp.asarray(_LANE_I)
    wot = Wo.T
    outs = []
    for n in range(N):
        vp, idx, wt = _prep(n, q[n], v[n], p8[n], wvt, bv2, wsxt, bsx,
                            wsyt, bsy, wat, ba2, selxy, lanef, lanei)
        table = _t4(vp)
        acc = _sc_gather(table, idx, wt)
        outs.append(_proj(acc.reshape(QN, DIM), wot, bo2))
    return jnp.stack(outs)
